# baseline (speedup 1.0000x reference)
import jax
import jax.numpy as jnp
from jax.experimental import pallas as pl

B = 4
S = 8192
D = 2048
STRIDE = 32
R = S // STRIDE
TOTAL = B * R

BLK = 128


def _copy_body(x_ref, o_ref):
    o_ref[...] = x_ref[...]


def kernel(x):
    x2 = x.reshape(TOTAL, STRIDE * D)
    y = pl.pallas_call(
        _copy_body,
        grid=(TOTAL // BLK,),
        in_specs=[pl.BlockSpec((BLK, D), lambda r: (r, 0))],
        out_specs=pl.BlockSpec((BLK, D), lambda r: (r, 0)),
        out_shape=jax.ShapeDtypeStruct((TOTAL, D), jnp.float32),
    )(x2)
    return y.reshape(B, R // 16, 16, D)
